# baseline (speedup 1.0000x reference)
import functools

import jax
import jax.numpy as jnp
from jax import lax
from jax.experimental import pallas as pl
from jax.experimental.pallas import tpu as pltpu
from jax.experimental.pallas import tpu_sc as plsc

VOCAB = 1000000
D = 64
B = 16384
NC, NS, L = 2, 16, 16
NW = NC * NS
BPW = B // NW
CHUNK = 128
NCHUNK = BPW // CHUNK
MAGIC = 12582912.0


def _sc_embed(x, w_flat, scales):
    mesh = plsc.VectorSubcoreMesh(core_axis_name="c", subcore_axis_name="s")

    @functools.partial(
        pl.kernel,
        mesh=mesh,
        out_type=jax.ShapeDtypeStruct((D, B), jnp.float32),
        scratch_types=[
            pltpu.VMEM((BPW,), jnp.int32),
            pltpu.VMEM((D, BPW), jnp.int32),
            pltpu.VMEM((D, BPW), jnp.float32),
            pltpu.VMEM((BPW,), jnp.float32),
            pltpu.SemaphoreType.DMA,
        ],
    )
    def k(x_hbm, w_hbm, s_hbm, out_hbm, idx_v, fidx_v, gath_v, sc_v, sem):
        wid = lax.axis_index("s") * NC + lax.axis_index("c")
        base = wid * BPW
        pltpu.sync_copy(x_hbm.at[pl.ds(base, BPW)], idx_v)
        scp = []
        for j in range(NCHUNK):
            sl = pl.ds(j * CHUNK, CHUNK)
            scp.append(pltpu.async_copy(s_hbm.at[idx_v.at[sl]], sc_v.at[sl], sem))

        def fire_body(d, carry):
            off = d * VOCAB
            for j in range(BPW // L):
                sl = pl.ds(j * L, L)
                fidx_v[d, sl] = idx_v[sl] + off
            for j in range(NCHUNK):
                sl = pl.ds(j * CHUNK, CHUNK)
                pltpu.async_copy(w_hbm.at[fidx_v.at[d, sl]], gath_v.at[d, sl], sem)
            return carry

        lax.fori_loop(0, D, fire_body, 0)

        for cp in scp:
            cp.wait()

        def drain_body(d, carry):
            for j in range(NCHUNK):
                sl = pl.ds(j * CHUNK, CHUNK)
                pltpu.make_async_copy(
                    w_hbm.at[fidx_v.at[d, sl]], gath_v.at[d, sl], sem
                ).wait()
            return carry

        lax.fori_loop(0, D, drain_body, 0)

        def comp_body(d, carry):
            for j in range(BPW // L):
                sl = pl.ds(j * L, L)
                v = gath_v[d, sl]
                v = (v + MAGIC) - MAGIC
                v = jnp.minimum(jnp.maximum(v, -128.0), 127.0)
                gath_v[d, sl] = v * sc_v[sl]
            return carry

        lax.fori_loop(0, D, comp_body, 0)
        pltpu.sync_copy(gath_v, out_hbm.at[:, pl.ds(base, BPW)])

    return k(x, w_flat, scales)


def kernel(x, weights, scales):
    w_flat = jnp.transpose(weights).reshape(VOCAB * D)
    out_t = _sc_embed(x.astype(jnp.int32), w_flat, scales)
    return jnp.transpose(out_t)


# speedup vs baseline: 5.6203x; 5.6203x over previous
import functools

import jax
import jax.numpy as jnp
from jax import lax
from jax.experimental import pallas as pl
from jax.experimental.pallas import tpu as pltpu
from jax.experimental.pallas import tpu_sc as plsc

VOCAB = 1000000
D = 64
B = 16384
NC, NS, L = 2, 16, 16
NW = NC * NS
BPW = B // NW
CHUNK = 128
NCHUNK = BPW // CHUNK
MAGIC = 12582912.0
CB = 2048


def _tc_relayout(w_t):
    nb = (VOCAB + CB - 1) // CB

    def body(wt_ref, out_ref):
        ident = jax.lax.broadcasted_iota(jnp.int32, (D, D), 0) == \
            jax.lax.broadcasted_iota(jnp.int32, (D, D), 1)
        out_ref[...] = jax.lax.dot_general(
            wt_ref[...],
            ident.astype(jnp.float32),
            (((0,), (0,)), ((), ())),
            preferred_element_type=jnp.float32,
        )

    return pl.pallas_call(
        body,
        grid=(nb,),
        in_specs=[pl.BlockSpec((D, CB), lambda i: (0, i))],
        out_specs=pl.BlockSpec((CB, D), lambda i: (i, 0)),
        out_shape=jax.ShapeDtypeStruct((VOCAB, D), jnp.float32),
    )(w_t)


def _sc_embed(x, weights, scales):
    mesh = plsc.VectorSubcoreMesh(core_axis_name="c", subcore_axis_name="s")

    @functools.partial(
        pl.kernel,
        mesh=mesh,
        out_type=jax.ShapeDtypeStruct((B, D), jnp.float32),
        compiler_params=pltpu.CompilerParams(use_tc_tiling_on_sc=False),
        scratch_types=[
            pltpu.VMEM((BPW,), jnp.int32),
            pltpu.VMEM((BPW, D), jnp.float32),
            pltpu.VMEM((BPW,), jnp.float32),
            pltpu.SemaphoreType.DMA,
        ],
    )
    def k(x_hbm, w_hbm, s_hbm, out_hbm, idx_v, rows_v, sc_v, sem):
        wid = lax.axis_index("s") * NC + lax.axis_index("c")
        base = wid * BPW
        pltpu.sync_copy(x_hbm.at[pl.ds(base, BPW)], idx_v)
        copies = []
        for j in range(NCHUNK):
            sl = pl.ds(j * CHUNK, CHUNK)
            copies.append(pltpu.async_copy(w_hbm.at[idx_v.at[sl]], rows_v.at[sl], sem))
            copies.append(pltpu.async_copy(s_hbm.at[idx_v.at[sl]], sc_v.at[sl], sem))
        for cp in copies:
            cp.wait()

        def grp_body(g, carry):
            r0 = g * L
            scg = sc_v[pl.ds(r0, L)]
            for i in range(L):
                sc = scg[i]
                for c in range(D // L):
                    v = rows_v[r0 + i, pl.ds(c * L, L)]
                    v = (v + MAGIC) - MAGIC
                    v = jnp.minimum(jnp.maximum(v, -128.0), 127.0)
                    rows_v[r0 + i, pl.ds(c * L, L)] = v * sc
            return carry

        lax.fori_loop(0, BPW // L, grp_body, 0)
        pltpu.sync_copy(rows_v, out_hbm.at[pl.ds(base, BPW)])

    return k(x, weights, scales)


def kernel(x, weights, scales):
    w_rows = _tc_relayout(jnp.transpose(weights))
    return _sc_embed(x.astype(jnp.int32), w_rows, scales)


# speedup vs baseline: 13.5007x; 2.4021x over previous
import functools

import jax
import jax.numpy as jnp
from jax import lax
from jax.experimental import pallas as pl
from jax.experimental.pallas import tpu as pltpu
from jax.experimental.pallas import tpu_sc as plsc

VOCAB = 1000000
D = 64
B = 16384
NC, NS, L = 2, 16, 16
NW = NC * NS
BPW = B // NW
CHUNK = 128
NCHUNK = BPW // CHUNK
MAGIC = 12582912.0
CB = 4096
HALF = CB // 2
NB = (VOCAB + CB - 1) // CB
ROWS = NB * HALF


def _tc_relayout(w_t):
    def body(wt_ref, out_ref):
        blk = wt_ref[...]
        out_ref[:, 0:D] = blk[:, 0:HALF].T
        out_ref[:, D:2 * D] = blk[:, HALF:CB].T

    return pl.pallas_call(
        body,
        grid=(NB,),
        in_specs=[pl.BlockSpec((D, CB), lambda i: (0, i))],
        out_specs=pl.BlockSpec((HALF, 2 * D), lambda i: (i, 0)),
        out_shape=jax.ShapeDtypeStruct((ROWS, 2 * D), jnp.float32),
    )(w_t)


def _sc_embed(x, w_rows, scales):
    mesh = plsc.VectorSubcoreMesh(core_axis_name="c", subcore_axis_name="s")

    @functools.partial(
        pl.kernel,
        mesh=mesh,
        out_type=jax.ShapeDtypeStruct((B, D), jnp.float32),
        compiler_params=pltpu.CompilerParams(use_tc_tiling_on_sc=False),
        scratch_types=[
            pltpu.VMEM((BPW,), jnp.int32),
            pltpu.VMEM((BPW,), jnp.int32),
            pltpu.VMEM((BPW, 2 * D), jnp.float32),
            pltpu.VMEM((BPW,), jnp.float32),
            pltpu.VMEM((BPW, D), jnp.float32),
            pltpu.SemaphoreType.DMA,
        ],
    )
    def k(x_hbm, w_hbm, s_hbm, out_hbm, idx_v, gidx_v, rows_v, sc_v, out_v, sem):
        wid = lax.axis_index("s") * NC + lax.axis_index("c")
        base = wid * BPW
        pltpu.sync_copy(x_hbm.at[pl.ds(base, BPW)], idx_v)
        for j in range(BPW // L):
            sl = pl.ds(j * L, L)
            v = idx_v[sl]
            gidx_v[sl] = lax.shift_left(lax.shift_right_logical(v, 12), 11) + \
                jnp.bitwise_and(v, 2047)
        copies = []
        for j in range(NCHUNK):
            sl = pl.ds(j * CHUNK, CHUNK)
            copies.append(pltpu.async_copy(w_hbm.at[gidx_v.at[sl]], rows_v.at[sl], sem))
            copies.append(pltpu.async_copy(s_hbm.at[idx_v.at[sl]], sc_v.at[sl], sem))
        for cp in copies:
            cp.wait()

        def grp_body(g, carry):
            r0 = g * L
            scg = sc_v[pl.ds(r0, L)]
            offg = lax.shift_left(
                jnp.bitwise_and(lax.shift_right_logical(idx_v[pl.ds(r0, L)], 11), 1), 6
            )
            for i in range(L):
                sc = scg[i]
                off = offg[i]
                for c in range(D // L):
                    v = rows_v[r0 + i, pl.ds(off + c * L, L)]
                    v = (v + MAGIC) - MAGIC
                    v = jnp.minimum(jnp.maximum(v, -128.0), 127.0)
                    out_v[r0 + i, pl.ds(c * L, L)] = v * sc
            return carry

        lax.fori_loop(0, BPW // L, grp_body, 0)
        pltpu.sync_copy(out_v, out_hbm.at[pl.ds(base, BPW)])

    return k(x, w_rows, scales)


def kernel(x, weights, scales):
    w_rows = _tc_relayout(jnp.transpose(weights))
    return _sc_embed(x.astype(jnp.int32), w_rows, scales)


# speedup vs baseline: 15.9636x; 1.1824x over previous
import functools

import jax
import jax.numpy as jnp
from jax import lax
from jax.experimental import pallas as pl
from jax.experimental.pallas import tpu as pltpu
from jax.experimental.pallas import tpu_sc as plsc

VOCAB = 1000000
D = 64
B = 16384
NC, NS, L = 2, 16, 16
NW = NC * NS
BPW = B // NW
CHUNK = 128
NCHUNK = BPW // CHUNK
MAGIC = 12582912.0
CB = 4096
HALF = CB // 2
NB = (VOCAB + CB - 1) // CB
ROWS = NB * HALF


def _tc_relayout(w_t):
    def body(wt_ref, out_ref):
        blk = wt_ref[...]
        stacked = jnp.concatenate([blk[:, 0:HALF], blk[:, HALF:CB]], axis=0)
        out_ref[...] = stacked.T

    return pl.pallas_call(
        body,
        grid=(NB,),
        in_specs=[pl.BlockSpec((D, CB), lambda i: (0, i))],
        out_specs=pl.BlockSpec((HALF, 2 * D), lambda i: (i, 0)),
        out_shape=jax.ShapeDtypeStruct((ROWS, 2 * D), jnp.float32),
    )(w_t)


def _sc_embed(x, w_rows, scales):
    mesh = plsc.VectorSubcoreMesh(core_axis_name="c", subcore_axis_name="s")

    @functools.partial(
        pl.kernel,
        mesh=mesh,
        out_type=jax.ShapeDtypeStruct((B, D), jnp.float32),
        compiler_params=pltpu.CompilerParams(use_tc_tiling_on_sc=False),
        scratch_types=[
            pltpu.VMEM((BPW,), jnp.int32),
            pltpu.VMEM((BPW,), jnp.int32),
            pltpu.VMEM((BPW, 2 * D), jnp.float32),
            pltpu.VMEM((BPW,), jnp.float32),
            pltpu.VMEM((BPW, D), jnp.float32),
            pltpu.SemaphoreType.DMA,
        ],
    )
    def k(x_hbm, w_hbm, s_hbm, out_hbm, idx_v, gidx_v, rows_v, sc_v, out_v, sem):
        wid = lax.axis_index("s") * NC + lax.axis_index("c")
        base = wid * BPW
        pltpu.sync_copy(x_hbm.at[pl.ds(base, BPW)], idx_v)
        for j in range(BPW // L):
            sl = pl.ds(j * L, L)
            v = idx_v[sl]
            gidx_v[sl] = lax.shift_left(lax.shift_right_logical(v, 12), 11) + \
                jnp.bitwise_and(v, 2047)
        copies = []
        for j in range(NCHUNK):
            sl = pl.ds(j * CHUNK, CHUNK)
            copies.append(pltpu.async_copy(w_hbm.at[gidx_v.at[sl]], rows_v.at[sl], sem))
            copies.append(pltpu.async_copy(s_hbm.at[idx_v.at[sl]], sc_v.at[sl], sem))
        for cp in copies:
            cp.wait()

        def grp_body(g, carry):
            r0 = g * L
            scg = sc_v[pl.ds(r0, L)]
            offg = lax.shift_left(
                jnp.bitwise_and(lax.shift_right_logical(idx_v[pl.ds(r0, L)], 11), 1), 6
            )
            for i in range(L):
                sc = scg[i]
                off = offg[i]
                for c in range(D // L):
                    v = rows_v[r0 + i, pl.ds(off + c * L, L)]
                    v = (v + MAGIC) - MAGIC
                    v = jnp.minimum(jnp.maximum(v, -128.0), 127.0)
                    out_v[r0 + i, pl.ds(c * L, L)] = v * sc
            return carry

        lax.fori_loop(0, BPW // L, grp_body, 0)
        pltpu.sync_copy(out_v, out_hbm.at[pl.ds(base, BPW)])

    return k(x, w_rows, scales)


def kernel(x, weights, scales):
    w_rows = _tc_relayout(jnp.transpose(weights))
    return _sc_embed(x.astype(jnp.int32), w_rows, scales)


# speedup vs baseline: 20.6267x; 1.2921x over previous
import functools

import jax
import jax.numpy as jnp
from jax import lax
from jax.experimental import pallas as pl
from jax.experimental.pallas import tpu as pltpu
from jax.experimental.pallas import tpu_sc as plsc

VOCAB = 1000000
D = 64
B = 16384
NC, NS, L = 2, 16, 16
NW = NC * NS
BPW = B // NW
CHUNK = 128
NCHUNK = BPW // CHUNK
MAGIC = 12582912.0
CB = 8192
HALF = CB // 2
NB = (VOCAB + CB - 1) // CB
ROWS = NB * HALF
SH_CB = CB.bit_length() - 1
SH_HF = SH_CB - 1


def _tc_relayout(w_t):
    def body(wt_ref, out_ref):
        blk = wt_ref[...]
        stacked = jnp.concatenate([blk[:, 0:HALF], blk[:, HALF:CB]], axis=0)
        out_ref[...] = stacked.T

    return pl.pallas_call(
        body,
        grid=(NB,),
        in_specs=[pl.BlockSpec((D, CB), lambda i: (0, i))],
        out_specs=pl.BlockSpec((HALF, 2 * D), lambda i: (i, 0)),
        out_shape=jax.ShapeDtypeStruct((ROWS, 2 * D), jnp.float32),
    )(w_t)


def _sc_embed(x, w_rows, scales):
    mesh = plsc.VectorSubcoreMesh(core_axis_name="c", subcore_axis_name="s")

    @functools.partial(
        pl.kernel,
        mesh=mesh,
        out_type=jax.ShapeDtypeStruct((B, D), jnp.float32),
        compiler_params=pltpu.CompilerParams(use_tc_tiling_on_sc=False),
        scratch_types=[
            pltpu.VMEM((BPW,), jnp.int32),
            pltpu.VMEM((BPW,), jnp.int32),
            pltpu.VMEM((BPW, 2 * D), jnp.float32),
            pltpu.VMEM((BPW,), jnp.float32),
            pltpu.VMEM((BPW, D), jnp.float32),
            pltpu.SemaphoreType.DMA,
        ],
    )
    def k(x_hbm, w_hbm, s_hbm, out_hbm, idx_v, gidx_v, rows_v, sc_v, out_v, sem):
        wid = lax.axis_index("s") * NC + lax.axis_index("c")
        base = wid * BPW
        pltpu.sync_copy(x_hbm.at[pl.ds(base, BPW)], idx_v)
        for j in range(BPW // L):
            sl = pl.ds(j * L, L)
            v = idx_v[sl]
            gidx_v[sl] = lax.shift_left(lax.shift_right_logical(v, SH_CB), SH_HF) + \
                jnp.bitwise_and(v, HALF - 1)
        copies = []
        for j in range(NCHUNK):
            sl = pl.ds(j * CHUNK, CHUNK)
            copies.append(pltpu.async_copy(w_hbm.at[gidx_v.at[sl]], rows_v.at[sl], sem))
            copies.append(pltpu.async_copy(s_hbm.at[idx_v.at[sl]], sc_v.at[sl], sem))
        for cp in copies:
            cp.wait()

        def grp_body(g, carry):
            r0 = g * L
            scg = sc_v[pl.ds(r0, L)]
            offg = lax.shift_left(
                jnp.bitwise_and(lax.shift_right_logical(idx_v[pl.ds(r0, L)], SH_HF), 1), 6
            )
            for i in range(L):
                sc = scg[i]
                off = offg[i]
                for c in range(D // L):
                    v = rows_v[r0 + i, pl.ds(off + c * L, L)]
                    v = (v + MAGIC) - MAGIC
                    v = jnp.minimum(jnp.maximum(v, -128.0), 127.0)
                    out_v[r0 + i, pl.ds(c * L, L)] = v * sc
            return carry

        lax.fori_loop(0, BPW // L, grp_body, 0)
        pltpu.sync_copy(out_v, out_hbm.at[pl.ds(base, BPW)])

    return k(x, w_rows, scales)


def kernel(x, weights, scales):
    w_rows = _tc_relayout(jnp.transpose(weights))
    return _sc_embed(x.astype(jnp.int32), w_rows, scales)


# speedup vs baseline: 23.2095x; 1.1252x over previous
import functools

import jax
import jax.numpy as jnp
from jax import lax
from jax.experimental import pallas as pl
from jax.experimental.pallas import tpu as pltpu
from jax.experimental.pallas import tpu_sc as plsc

VOCAB = 1000000
D = 64
B = 16384
NC, NS, L = 2, 16, 16
NW = NC * NS
BPW = B // NW
CHUNK = 128
NCHUNK = BPW // CHUNK
MAGIC = 12582912.0
CB = 16384
HALF = CB // 2
NB = (VOCAB + CB - 1) // CB
ROWS = NB * HALF
SH_CB = CB.bit_length() - 1
SH_HF = SH_CB - 1


def _tc_relayout(w_t):
    def body(wt_ref, out_ref):
        blk = wt_ref[...]
        stacked = jnp.concatenate([blk[:, 0:HALF], blk[:, HALF:CB]], axis=0)
        out_ref[...] = stacked.T

    return pl.pallas_call(
        body,
        grid=(NB,),
        in_specs=[pl.BlockSpec((D, CB), lambda i: (0, i))],
        out_specs=pl.BlockSpec((HALF, 2 * D), lambda i: (i, 0)),
        out_shape=jax.ShapeDtypeStruct((ROWS, 2 * D), jnp.float32),
    )(w_t)


def _sc_embed(x, w_rows, scales):
    mesh = plsc.VectorSubcoreMesh(core_axis_name="c", subcore_axis_name="s")

    @functools.partial(
        pl.kernel,
        mesh=mesh,
        out_type=jax.ShapeDtypeStruct((B, D), jnp.float32),
        compiler_params=pltpu.CompilerParams(use_tc_tiling_on_sc=False),
        scratch_types=[
            pltpu.VMEM((BPW,), jnp.int32),
            pltpu.VMEM((BPW,), jnp.int32),
            pltpu.VMEM((BPW, 2 * D), jnp.float32),
            pltpu.VMEM((BPW,), jnp.float32),
            pltpu.VMEM((BPW, D), jnp.float32),
            pltpu.SemaphoreType.DMA,
        ],
    )
    def k(x_hbm, w_hbm, s_hbm, out_hbm, idx_v, gidx_v, rows_v, sc_v, out_v, sem):
        wid = lax.axis_index("s") * NC + lax.axis_index("c")
        base = wid * BPW
        pltpu.sync_copy(x_hbm.at[pl.ds(base, BPW)], idx_v)
        for j in range(BPW // L):
            sl = pl.ds(j * L, L)
            v = idx_v[sl]
            gidx_v[sl] = lax.shift_left(lax.shift_right_logical(v, SH_CB), SH_HF) + \
                jnp.bitwise_and(v, HALF - 1)
        copies = []
        for j in range(NCHUNK):
            sl = pl.ds(j * CHUNK, CHUNK)
            copies.append(pltpu.async_copy(w_hbm.at[gidx_v.at[sl]], rows_v.at[sl], sem))
            copies.append(pltpu.async_copy(s_hbm.at[idx_v.at[sl]], sc_v.at[sl], sem))
        for cp in copies:
            cp.wait()

        def grp_body(g, carry):
            r0 = g * L
            scg = sc_v[pl.ds(r0, L)]
            offg = lax.shift_left(
                jnp.bitwise_and(lax.shift_right_logical(idx_v[pl.ds(r0, L)], SH_HF), 1), 6
            )
            for i in range(L):
                sc = scg[i]
                off = offg[i]
                for c in range(D // L):
                    v = rows_v[r0 + i, pl.ds(off + c * L, L)]
                    v = (v + MAGIC) - MAGIC
                    v = jnp.minimum(jnp.maximum(v, -128.0), 127.0)
                    out_v[r0 + i, pl.ds(c * L, L)] = v * sc
            return carry

        lax.fori_loop(0, BPW // L, grp_body, 0)
        pltpu.sync_copy(out_v, out_hbm.at[pl.ds(base, BPW)])

    return k(x, w_rows, scales)


def kernel(x, weights, scales):
    w_rows = _tc_relayout(jnp.transpose(weights))
    return _sc_embed(x.astype(jnp.int32), w_rows, scales)


# speedup vs baseline: 23.7622x; 1.0238x over previous
import functools

import jax
import jax.numpy as jnp
from jax import lax
from jax.experimental import pallas as pl
from jax.experimental.pallas import tpu as pltpu
from jax.experimental.pallas import tpu_sc as plsc

VOCAB = 1000000
D = 64
B = 16384
NC, NS, L = 2, 16, 16
NW = NC * NS
BPW = B // NW
CHUNK = 128
NCHUNK = BPW // CHUNK
MAGIC = 12582912.0
CB = 32768
HALF = CB // 2
NB = (VOCAB + CB - 1) // CB
ROWS = NB * HALF
SH_CB = CB.bit_length() - 1
SH_HF = SH_CB - 1


def _tc_relayout(w_t):
    def body(wt_ref, out_ref):
        blk = wt_ref[...]
        stacked = jnp.concatenate([blk[:, 0:HALF], blk[:, HALF:CB]], axis=0)
        out_ref[...] = stacked.T

    return pl.pallas_call(
        body,
        grid=(NB,),
        in_specs=[pl.BlockSpec((D, CB), lambda i: (0, i))],
        out_specs=pl.BlockSpec((HALF, 2 * D), lambda i: (i, 0)),
        out_shape=jax.ShapeDtypeStruct((ROWS, 2 * D), jnp.float32),
    )(w_t)


def _sc_embed(x, w_rows, scales):
    mesh = plsc.VectorSubcoreMesh(core_axis_name="c", subcore_axis_name="s")

    @functools.partial(
        pl.kernel,
        mesh=mesh,
        out_type=jax.ShapeDtypeStruct((B, D), jnp.float32),
        compiler_params=pltpu.CompilerParams(use_tc_tiling_on_sc=False),
        scratch_types=[
            pltpu.VMEM((BPW,), jnp.int32),
            pltpu.VMEM((BPW,), jnp.int32),
            pltpu.VMEM((BPW, 2 * D), jnp.float32),
            pltpu.VMEM((BPW,), jnp.float32),
            pltpu.VMEM((BPW, D), jnp.float32),
            pltpu.SemaphoreType.DMA,
        ],
    )
    def k(x_hbm, w_hbm, s_hbm, out_hbm, idx_v, gidx_v, rows_v, sc_v, out_v, sem):
        wid = lax.axis_index("s") * NC + lax.axis_index("c")
        base = wid * BPW
        pltpu.sync_copy(x_hbm.at[pl.ds(base, BPW)], idx_v)
        for j in range(BPW // L):
            sl = pl.ds(j * L, L)
            v = idx_v[sl]
            gidx_v[sl] = lax.shift_left(lax.shift_right_logical(v, SH_CB), SH_HF) + \
                jnp.bitwise_and(v, HALF - 1)
        copies = []
        for j in range(NCHUNK):
            sl = pl.ds(j * CHUNK, CHUNK)
            copies.append(pltpu.async_copy(w_hbm.at[gidx_v.at[sl]], rows_v.at[sl], sem))
            copies.append(pltpu.async_copy(s_hbm.at[idx_v.at[sl]], sc_v.at[sl], sem))
        for cp in copies:
            cp.wait()

        def grp_body(g, carry):
            r0 = g * L
            scg = sc_v[pl.ds(r0, L)]
            offg = lax.shift_left(
                jnp.bitwise_and(lax.shift_right_logical(idx_v[pl.ds(r0, L)], SH_HF), 1), 6
            )
            for i in range(L):
                sc = scg[i]
                off = offg[i]
                for c in range(D // L):
                    v = rows_v[r0 + i, pl.ds(off + c * L, L)]
                    v = (v + MAGIC) - MAGIC
                    v = jnp.minimum(jnp.maximum(v, -128.0), 127.0)
                    out_v[r0 + i, pl.ds(c * L, L)] = v * sc
            return carry

        lax.fori_loop(0, BPW // L, grp_body, 0)
        pltpu.sync_copy(out_v, out_hbm.at[pl.ds(base, BPW)])

    return k(x, w_rows, scales)


def kernel(x, weights, scales):
    w_rows = _tc_relayout(jnp.transpose(weights))
    return _sc_embed(x.astype(jnp.int32), w_rows, scales)
